# Initial kernel scaffold; baseline (speedup 1.0000x reference)
import jax
import jax.numpy as jnp
from jax.experimental import pallas as pl


def kernel(x, adj, W1, b1, alpha1, W2, b2, alpha2):
    raise NotImplementedError("write your pallas kernel here")


# speedup vs baseline: 12.7537x; 12.7537x over previous
import functools

import jax
import jax.numpy as jnp
from jax import lax
from jax.experimental import pallas as pl
from jax.experimental.pallas import tpu as pltpu, tpu_sc as plsc

_L = 128
_DEGW = 8


def _zero_vmem_2d(buf, rows, cols):
    zero16 = jnp.zeros((16,), jnp.float32)
    per_row = cols // 16

    def body(r, _):
        for q in range(per_row):
            buf[r, pl.ds(q * 16, 16)] = zero16
        return 0

    lax.fori_loop(0, rows, body, 0)


def _fill_vmem_2d(buf, rows, cols, value):
    v16 = jnp.full((16,), value, jnp.float32)
    per_row = cols // 16

    def body(r, _):
        for q in range(per_row):
            buf[r, pl.ds(q * 16, 16)] = v16
        return 0

    lax.fori_loop(0, rows, body, 0)


def _copy_zero_slice(zbuf, dst, base, rows):
    nfull, rem = rows // 128, rows % 128
    for q in range(nfull):
        pltpu.sync_copy(zbuf, dst.at[pl.ds(base + q * 128, 128)])
    if rem:
        pltpu.sync_copy(zbuf.at[pl.ds(0, rem)],
                        dst.at[pl.ds(base + nfull * 128, rem)])


def _make_deg_kernel(n_pad, k_chunks, rows_per_tile, nc, ns):
    mesh = plsc.VectorSubcoreMesh(core_axis_name="c", subcore_axis_name="s")

    @functools.partial(
        pl.kernel,
        out_type=jax.ShapeDtypeStruct((nc, n_pad, _DEGW), jnp.float32),
        mesh=mesh,
        scratch_types=[
            pltpu.VMEM((k_chunks, _L), jnp.int32),
            pltpu.VMEM((128, _DEGW), jnp.float32),
            pltpu.VMEM((_L, _DEGW), jnp.float32),
            pltpu.VMEM_SHARED((n_pad, _DEGW), jnp.float32),
        ],
    )
    def deg_kernel(dstp_hbm, out_hbm, dst_v, zbuf, ones_v, d_sh):
        c = lax.axis_index("c")
        s = lax.axis_index("s")
        t = c * ns + s
        pltpu.sync_copy(dstp_hbm.at[t], dst_v)
        _zero_vmem_2d(zbuf, 128, _DEGW)
        _fill_vmem_2d(ones_v, _L, _DEGW, 1.0)
        base = s * rows_per_tile
        _copy_zero_slice(zbuf, d_sh, base, rows_per_tile)
        plsc.subcore_barrier()

        def body(j, _):
            pltpu.sync_copy(ones_v, d_sh.at[dst_v.at[j]], add=True)
            return 0

        lax.fori_loop(0, k_chunks, body, 0)
        plsc.subcore_barrier()
        pltpu.sync_copy(d_sh.at[pl.ds(base, rows_per_tile)],
                        out_hbm.at[c, pl.ds(base, rows_per_tile)])

    return deg_kernel


def _make_agg_kernel(n_pad, d, k_chunks, rows_per_tile, nc, ns):
    mesh = plsc.VectorSubcoreMesh(core_axis_name="c", subcore_axis_name="s")

    @functools.partial(
        pl.kernel,
        out_type=jax.ShapeDtypeStruct((nc, n_pad, d), jnp.float32),
        mesh=mesh,
        scratch_types=[
            pltpu.VMEM((k_chunks, _L), jnp.int32),
            pltpu.VMEM((k_chunks, _L), jnp.int32),
            pltpu.VMEM((_L, d), jnp.float32),
            pltpu.VMEM_SHARED((n_pad, d), jnp.float32),
            pltpu.SemaphoreType.DMA,
        ],
    )
    def agg_kernel(y_hbm, srcp_hbm, dstp_hbm, out_hbm,
                   src_v, dst_v, buf, z_sh, sem):
        c = lax.axis_index("c")
        s = lax.axis_index("s")
        t = c * ns + s
        pltpu.sync_copy(srcp_hbm.at[t], src_v)
        pltpu.sync_copy(dstp_hbm.at[t], dst_v)
        _zero_vmem_2d(buf, _L, d)
        base = s * rows_per_tile
        _copy_zero_slice(buf, z_sh, base, rows_per_tile)
        plsc.subcore_barrier()

        def body(j, _):
            pltpu.async_copy(y_hbm.at[src_v.at[j]], buf, sem).wait()
            pltpu.sync_copy(buf, z_sh.at[dst_v.at[j]], add=True)
            return 0

        lax.fori_loop(0, k_chunks, body, 0)
        plsc.subcore_barrier()
        pltpu.sync_copy(z_sh.at[pl.ds(base, rows_per_tile)],
                        out_hbm.at[c, pl.ds(base, rows_per_tile)])

    return agg_kernel


def _tc_pre(x, w1, dp0, dp1, n, d, br):
    def body(x_ref, w_ref, d0_ref, d1_ref, dis_ref, y_ref):
        deg = d0_ref[...] + d1_ref[...] + 1.0
        dis = lax.rsqrt(deg)
        dis_ref[...] = dis
        xw = lax.dot_general(x_ref[...], w_ref[...], (((1,), (1,)), ((), ())),
                             preferred_element_type=jnp.float32)
        y_ref[...] = xw * dis

    return pl.pallas_call(
        body,
        grid=(n // br,),
        in_specs=[
            pl.BlockSpec((br, d), lambda i: (i, 0)),
            pl.BlockSpec((d, d), lambda i: (0, 0)),
            pl.BlockSpec((br, 1), lambda i: (i, 0)),
            pl.BlockSpec((br, 1), lambda i: (i, 0)),
        ],
        out_specs=[
            pl.BlockSpec((br, 1), lambda i: (i, 0)),
            pl.BlockSpec((br, d), lambda i: (i, 0)),
        ],
        out_shape=[
            jax.ShapeDtypeStruct((n, 1), jnp.float32),
            jax.ShapeDtypeStruct((n, d), jnp.float32),
        ],
    )(x, w1, dp0, dp1)


def _tc_mid(zp0, zp1, y1, dis, b1, a1, w2, n, d, br):
    def body(z0_ref, z1_ref, y_ref, dis_ref, b_ref, a_ref, w_ref, y2_ref):
        dis = dis_ref[...]
        t = (z0_ref[...] + z1_ref[...] + y_ref[...]) * dis + b_ref[...]
        h = jnp.where(t >= 0.0, t, a_ref[0, 0] * t)
        hw = lax.dot_general(h, w_ref[...], (((1,), (1,)), ((), ())),
                             preferred_element_type=jnp.float32)
        y2_ref[...] = hw * dis

    return pl.pallas_call(
        body,
        grid=(n // br,),
        in_specs=[
            pl.BlockSpec((br, d), lambda i: (i, 0)),
            pl.BlockSpec((br, d), lambda i: (i, 0)),
            pl.BlockSpec((br, d), lambda i: (i, 0)),
            pl.BlockSpec((br, 1), lambda i: (i, 0)),
            pl.BlockSpec((1, d), lambda i: (0, 0)),
            pl.BlockSpec((1, 1), lambda i: (0, 0)),
            pl.BlockSpec((d, d), lambda i: (0, 0)),
        ],
        out_specs=pl.BlockSpec((br, d), lambda i: (i, 0)),
        out_shape=jax.ShapeDtypeStruct((n, d), jnp.float32),
    )(zp0, zp1, y1, dis, b1, a1, w2)


def _tc_post(zp0, zp1, y2, dis, b2, a2, n, d, br):
    def body(z0_ref, z1_ref, y_ref, dis_ref, b_ref, a_ref, o_ref):
        t = (z0_ref[...] + z1_ref[...] + y_ref[...]) * dis_ref[...] + b_ref[...]
        o_ref[...] = jnp.where(t >= 0.0, t, a_ref[0, 0] * t)

    return pl.pallas_call(
        body,
        grid=(n // br,),
        in_specs=[
            pl.BlockSpec((br, d), lambda i: (i, 0)),
            pl.BlockSpec((br, d), lambda i: (i, 0)),
            pl.BlockSpec((br, d), lambda i: (i, 0)),
            pl.BlockSpec((br, 1), lambda i: (i, 0)),
            pl.BlockSpec((1, d), lambda i: (0, 0)),
            pl.BlockSpec((1, 1), lambda i: (0, 0)),
        ],
        out_specs=pl.BlockSpec((br, d), lambda i: (i, 0)),
        out_shape=jax.ShapeDtypeStruct((n, d), jnp.float32),
    )(zp0, zp1, y2, dis, b2, a2)


def kernel(x, adj, W1, b1, alpha1, W2, b2, alpha2):
    n, d = x.shape
    e = adj.shape[1]
    info = plsc.get_sparse_core_info()
    nc, ns = info.num_cores, info.num_subcores
    nw = nc * ns

    per_tile = -(-e // nw)
    k_chunks = -(-per_tile // _L)
    e_pad = nw * k_chunks * _L
    rows_per_tile = -(-(n + 1) // (ns * 8)) * 8
    n_pad = rows_per_tile * ns

    src = jnp.concatenate(
        [adj[0], jnp.zeros((e_pad - e,), adj.dtype)]).reshape(nw, k_chunks, _L)
    dst = jnp.concatenate(
        [adj[1], jnp.full((e_pad - e,), n, adj.dtype)]).reshape(nw, k_chunks, _L)

    degp = _make_deg_kernel(n_pad, k_chunks, rows_per_tile, nc, ns)(dst)
    dp0 = degp[0, :n, 0:1]
    dp1 = degp[1, :n, 0:1]

    br = 1000 if n % 1000 == 0 else 8
    b1r = b1.reshape(1, d)
    b2r = b2.reshape(1, d)
    a1r = alpha1.reshape(1, 1)
    a2r = alpha2.reshape(1, 1)

    dis, y1 = _tc_pre(x, W1, dp0, dp1, n, d, br)
    z1 = _make_agg_kernel(n_pad, d, k_chunks, rows_per_tile, nc, ns)(y1, src, dst)
    y2 = _tc_mid(z1[0, :n], z1[1, :n], y1, dis, b1r, a1r, W2, n, d, br)

    z2 = _make_agg_kernel(n_pad, d, k_chunks, rows_per_tile, nc, ns)(y2, src, dst)
    return _tc_post(z2[0, :n], z2[1, :n], y2, dis, b2r, a2r, n, d, br)
